# baseline (speedup 1.0000x reference)
import functools

import jax
import jax.numpy as jnp
from jax import lax
from jax.experimental import pallas as pl
from jax.experimental.pallas import tpu as pltpu
from jax.experimental.pallas import tpu_sc as plsc

VOCAB = 1000
D_MODEL = 1024
VPAD = 1024
LSE_COL = VOCAB
BATCH = 16
SEQ = 2048
N_POS = BATCH * (SEQ - 1)
N_PAD = BATCH * SEQ

NC, NS, L = 2, 16, 16
NW = NC * NS
PER_W = N_PAD // NW
N_GATHER = PER_W // 128


def _idx_body(x_ref, im_ref, il_ref):
    xv = x_ref[...]
    t = jnp.concatenate([xv[:, 1:], xv[:, :1]], axis=1)
    row = xv * VPAD
    im_ref[...] = row + t
    il_ref[...] = row + LSE_COL


_idx = pl.pallas_call(
    _idx_body,
    out_shape=(
        jax.ShapeDtypeStruct((BATCH, SEQ), jnp.int32),
        jax.ShapeDtypeStruct((BATCH, SEQ), jnp.int32),
    ),
)


def _mm_lse_body(emb_ref, w_ref, m_ref):
    m = jnp.dot(emb_ref[...], w_ref[...], preferred_element_type=jnp.float32)
    col = lax.broadcasted_iota(jnp.int32, (VPAD, VPAD), 1)
    valid = col < VOCAB
    mx = jnp.max(jnp.where(valid, m, -jnp.inf), axis=1, keepdims=True)
    s = jnp.sum(jnp.where(valid, jnp.exp(m - mx), 0.0), axis=1, keepdims=True)
    lse = jnp.log(s) + mx
    m_ref[...] = jnp.where(col == LSE_COL, lse, m)


_mm_lse = pl.pallas_call(
    _mm_lse_body,
    out_shape=jax.ShapeDtypeStruct((VPAD, VPAD), jnp.float32),
)


@functools.cache
def _get_gather_nll():
    mesh = plsc.VectorSubcoreMesh(
        core_axis_name="c", subcore_axis_name="s", num_cores=NC)

    @functools.partial(
        pl.kernel,
        mesh=mesh,
        out_type=jax.ShapeDtypeStruct((NW, L), jnp.float32),
        scratch_types=[
            pltpu.VMEM((PER_W,), jnp.int32),
            pltpu.VMEM((PER_W,), jnp.int32),
            pltpu.VMEM((PER_W,), jnp.float32),
            pltpu.VMEM((PER_W,), jnp.float32),
            pltpu.VMEM((L,), jnp.float32),
            pltpu.SemaphoreType.DMA,
        ],
    )
    def _gather_nll(m_hbm, im_hbm, il_hbm, part_hbm,
                    idx_m, idx_l, val_m, val_l, accv, sem):
        wid = lax.axis_index("s") * NC + lax.axis_index("c")
        base = wid * PER_W
        pltpu.sync_copy(im_hbm.at[pl.ds(base, PER_W)], idx_m)
        pltpu.sync_copy(il_hbm.at[pl.ds(base, PER_W)], idx_l)

        copies = []
        for j in range(N_GATHER):
            sl = pl.ds(j * 128, 128)
            copies.append(
                pltpu.async_copy(m_hbm.at[idx_m.at[sl]], val_m.at[sl], sem))
            copies.append(
                pltpu.async_copy(m_hbm.at[idx_l.at[sl]], val_l.at[sl], sem))
        for cp in copies:
            cp.wait()

        def accum(c, acc):
            o = c * L
            gid = base + o + lax.iota(jnp.int32, L)
            d = val_l[pl.ds(o, L)] - val_m[pl.ds(o, L)]
            return acc + jnp.where((gid & (SEQ - 1)) != SEQ - 1, d, 0.0)

        acc = lax.fori_loop(0, PER_W // L, accum, jnp.zeros((L,), jnp.float32))
        accv[...] = acc
        pltpu.sync_copy(accv, part_hbm.at[wid])

    return _gather_nll


def _reduce_body(p_ref, out_ref):
    out_ref[0, 0] = jnp.sum(p_ref[...]) * (1.0 / N_POS)


_reduce = pl.pallas_call(
    _reduce_body,
    out_shape=jax.ShapeDtypeStruct((1, 1), jnp.float32),
    out_specs=pl.BlockSpec(memory_space=pltpu.SMEM),
)


def kernel(emb, w_out, x):
    emb_pad = jnp.pad(emb, ((0, VPAD - VOCAB), (0, 0))).astype(jnp.bfloat16)
    w_pad = jnp.pad(w_out, ((0, 0), (0, VPAD - VOCAB))).astype(jnp.bfloat16)

    idx_m, idx_l = _idx(x.astype(jnp.int32))
    m_aug = _mm_lse(emb_pad, w_pad)
    partials = _get_gather_nll()(
        m_aug.reshape(-1), idx_m.reshape(-1), idx_l.reshape(-1))
    return _reduce(partials)[0, 0]


# speedup vs baseline: 1.1342x; 1.1342x over previous
import functools

import jax
import jax.numpy as jnp
from jax import lax
from jax.experimental import pallas as pl
from jax.experimental.pallas import tpu as pltpu
from jax.experimental.pallas import tpu_sc as plsc

VOCAB = 1000
D_MODEL = 1024
BATCH = 16
SEQ = 2048
N_POS = BATCH * (SEQ - 1)
N_PAD = BATCH * SEQ

NC, NS, L = 2, 16, 16
NW = NC * NS
PER_W = N_PAD // NW
N_GATHER = PER_W // 128


def _prep_body(emb_ref, w_ref, x_ref, m_ref, lse_ref, im_ref, il_ref):
    a = emb_ref[...].astype(jnp.bfloat16)
    b = w_ref[...].astype(jnp.bfloat16)
    m = jnp.dot(a, b, preferred_element_type=jnp.float32)
    mx = jnp.max(m, axis=1, keepdims=True)
    s = jnp.sum(jnp.exp(m - mx), axis=1, keepdims=True)
    lse = jnp.log(s) + mx
    m_ref[...] = m
    lse_ref[...] = jnp.broadcast_to(lse, (VOCAB, 128))
    xv = x_ref[...]
    t = jnp.concatenate([xv[:, 1:], xv[:, :1]], axis=1)
    im_ref[...] = xv * VOCAB + t
    il_ref[...] = xv * 128


_prep = pl.pallas_call(
    _prep_body,
    out_shape=(
        jax.ShapeDtypeStruct((VOCAB, VOCAB), jnp.float32),
        jax.ShapeDtypeStruct((VOCAB, 128), jnp.float32),
        jax.ShapeDtypeStruct((BATCH, SEQ), jnp.int32),
        jax.ShapeDtypeStruct((BATCH, SEQ), jnp.int32),
    ),
)


@functools.cache
def _get_gather_nll():
    mesh = plsc.VectorSubcoreMesh(
        core_axis_name="c", subcore_axis_name="s", num_cores=NC)

    @functools.partial(
        pl.kernel,
        mesh=mesh,
        out_type=jax.ShapeDtypeStruct((NW, L), jnp.float32),
        scratch_types=[
            pltpu.VMEM((PER_W,), jnp.int32),
            pltpu.VMEM((PER_W,), jnp.int32),
            pltpu.VMEM((PER_W,), jnp.float32),
            pltpu.VMEM((PER_W,), jnp.float32),
            pltpu.VMEM((L,), jnp.float32),
            pltpu.SemaphoreType.DMA,
        ],
    )
    def _gather_nll(m_hbm, lse_hbm, im_hbm, il_hbm, part_hbm,
                    idx_m, idx_l, val_m, val_l, accv, sem):
        wid = lax.axis_index("s") * NC + lax.axis_index("c")
        base = wid * PER_W
        pltpu.sync_copy(im_hbm.at[pl.ds(base, PER_W)], idx_m)
        pltpu.sync_copy(il_hbm.at[pl.ds(base, PER_W)], idx_l)

        copies = []
        for j in range(N_GATHER):
            sl = pl.ds(j * 128, 128)
            copies.append(
                pltpu.async_copy(m_hbm.at[idx_m.at[sl]], val_m.at[sl], sem))
            copies.append(
                pltpu.async_copy(lse_hbm.at[idx_l.at[sl]], val_l.at[sl], sem))
        for cp in copies:
            cp.wait()

        def accum(c, acc):
            o = c * L
            gid = base + o + lax.iota(jnp.int32, L)
            d = val_l[pl.ds(o, L)] - val_m[pl.ds(o, L)]
            return acc + jnp.where((gid & (SEQ - 1)) != SEQ - 1, d, 0.0)

        acc = lax.fori_loop(0, PER_W // L, accum, jnp.zeros((L,), jnp.float32))
        accv[...] = acc
        pltpu.sync_copy(accv, part_hbm.at[wid])

    return _gather_nll


def _reduce_body(p_ref, out_ref):
    out_ref[0, 0] = jnp.sum(p_ref[...]) * (1.0 / N_POS)


_reduce = pl.pallas_call(
    _reduce_body,
    out_shape=jax.ShapeDtypeStruct((1, 1), jnp.float32),
    out_specs=pl.BlockSpec(memory_space=pltpu.SMEM),
)


def kernel(emb, w_out, x):
    m, lse_b, idx_m, idx_l = _prep(emb, w_out, x.astype(jnp.int32))
    partials = _get_gather_nll()(
        m.reshape(-1), lse_b.reshape(-1), idx_m.reshape(-1), idx_l.reshape(-1))
    return _reduce(partials)[0, 0]


# speedup vs baseline: 1.1976x; 1.0559x over previous
import functools

import jax
import jax.numpy as jnp
from jax import lax
from jax.experimental import pallas as pl
from jax.experimental.pallas import tpu as pltpu
from jax.experimental.pallas import tpu_sc as plsc

VOCAB = 1000
D_MODEL = 1024
BATCH = 16
SEQ = 2048
N_POS = BATCH * (SEQ - 1)
N_PAD = BATCH * SEQ

NC, NS, L = 2, 16, 16
NW = NC * NS
PER_W = N_PAD // NW
N_GATHER = PER_W // 128


def _prep_body(emb_ref, w_ref, x_ref, d_ref, im_ref):
    a = emb_ref[...].astype(jnp.bfloat16)
    b = w_ref[...].astype(jnp.bfloat16)
    m = jnp.dot(a, b, preferred_element_type=jnp.float32)
    mx = jnp.max(m, axis=1, keepdims=True)
    s = jnp.sum(jnp.exp(m - mx), axis=1, keepdims=True)
    lse = jnp.log(s) + mx
    d_ref[...] = lse - m
    xv = x_ref[...]
    t = jnp.concatenate([xv[:, 1:], xv[:, :1]], axis=1)
    im_ref[...] = xv * VOCAB + t


_prep = pl.pallas_call(
    _prep_body,
    out_shape=(
        jax.ShapeDtypeStruct((VOCAB, VOCAB), jnp.float32),
        jax.ShapeDtypeStruct((BATCH, SEQ), jnp.int32),
    ),
)


@functools.cache
def _get_gather_nll():
    mesh = plsc.VectorSubcoreMesh(
        core_axis_name="c", subcore_axis_name="s", num_cores=NC)

    @functools.partial(
        pl.kernel,
        mesh=mesh,
        out_type=jax.ShapeDtypeStruct((NW, L), jnp.float32),
        scratch_types=[
            pltpu.VMEM((PER_W,), jnp.int32),
            pltpu.VMEM((PER_W,), jnp.float32),
            pltpu.VMEM((L,), jnp.float32),
            pltpu.SemaphoreType.DMA,
        ],
    )
    def _gather_nll(d_hbm, im_hbm, part_hbm, idx_m, val_m, accv, sem):
        wid = lax.axis_index("s") * NC + lax.axis_index("c")
        base = wid * PER_W
        pltpu.sync_copy(im_hbm.at[pl.ds(base, PER_W)], idx_m)

        copies = []
        for j in range(N_GATHER):
            sl = pl.ds(j * 128, 128)
            copies.append(
                pltpu.async_copy(d_hbm.at[idx_m.at[sl]], val_m.at[sl], sem))
        for cp in copies:
            cp.wait()

        def accum(c, acc):
            o = c * L
            gid = base + o + lax.iota(jnp.int32, L)
            d = val_m[pl.ds(o, L)]
            return acc + jnp.where((gid & (SEQ - 1)) != SEQ - 1, d, 0.0)

        acc = lax.fori_loop(0, PER_W // L, accum, jnp.zeros((L,), jnp.float32))
        accv[...] = acc
        pltpu.sync_copy(accv, part_hbm.at[wid])

    return _gather_nll


def _reduce_body(p_ref, out_ref):
    out_ref[0, 0] = jnp.sum(p_ref[...]) * (1.0 / N_POS)


_reduce = pl.pallas_call(
    _reduce_body,
    out_shape=jax.ShapeDtypeStruct((1, 1), jnp.float32),
    out_specs=pl.BlockSpec(memory_space=pltpu.SMEM),
)


def kernel(emb, w_out, x):
    d, idx_m = _prep(emb, w_out, x.astype(jnp.int32))
    partials = _get_gather_nll()(d.reshape(-1), idx_m.reshape(-1))
    return _reduce(partials)[0, 0]


# speedup vs baseline: 1.2014x; 1.0032x over previous
import functools

import jax
import jax.numpy as jnp
from jax import lax
from jax.experimental import pallas as pl
from jax.experimental.pallas import tpu as pltpu
from jax.experimental.pallas import tpu_sc as plsc

VOCAB = 1000
D_MODEL = 1024
BATCH = 16
SEQ = 2048
N_POS = BATCH * (SEQ - 1)
N_PAD = BATCH * SEQ

NC, NS, L = 1, 16, 16
NW = NC * NS
PER_W = N_PAD // NW
N_GATHER = PER_W // 128


def _prep_body(emb_ref, w_ref, x_ref, d_ref, im_ref):
    a = emb_ref[...].astype(jnp.bfloat16)
    b = w_ref[...].astype(jnp.bfloat16)
    m = jnp.dot(a, b, preferred_element_type=jnp.float32)
    mx = jnp.max(m, axis=1, keepdims=True)
    s = jnp.sum(jnp.exp(m - mx), axis=1, keepdims=True)
    lse = jnp.log(s) + mx
    d_ref[...] = lse - m
    xv = x_ref[...]
    t = jnp.concatenate([xv[:, 1:], xv[:, :1]], axis=1)
    im_ref[...] = xv * VOCAB + t


_prep = pl.pallas_call(
    _prep_body,
    out_shape=(
        jax.ShapeDtypeStruct((VOCAB, VOCAB), jnp.float32),
        jax.ShapeDtypeStruct((BATCH, SEQ), jnp.int32),
    ),
)


@functools.cache
def _get_gather_nll():
    mesh = plsc.VectorSubcoreMesh(
        core_axis_name="c", subcore_axis_name="s", num_cores=NC)

    @functools.partial(
        pl.kernel,
        mesh=mesh,
        out_type=jax.ShapeDtypeStruct((NW, L), jnp.float32),
        scratch_types=[
            pltpu.VMEM((PER_W,), jnp.int32),
            pltpu.VMEM((PER_W,), jnp.float32),
            pltpu.VMEM((L,), jnp.float32),
            pltpu.SemaphoreType.DMA,
        ],
    )
    def _gather_nll(d_hbm, im_hbm, part_hbm, idx_m, val_m, accv, sem):
        wid = lax.axis_index("s") * NC + lax.axis_index("c")
        base = wid * PER_W
        pltpu.sync_copy(im_hbm.at[pl.ds(base, PER_W)], idx_m)

        copies = []
        for j in range(N_GATHER):
            sl = pl.ds(j * 128, 128)
            copies.append(
                pltpu.async_copy(d_hbm.at[idx_m.at[sl]], val_m.at[sl], sem))
        for cp in copies:
            cp.wait()

        def accum(c, acc):
            o = c * L
            gid = base + o + lax.iota(jnp.int32, L)
            d = val_m[pl.ds(o, L)]
            return acc + jnp.where((gid & (SEQ - 1)) != SEQ - 1, d, 0.0)

        acc = lax.fori_loop(0, PER_W // L, accum, jnp.zeros((L,), jnp.float32))
        accv[...] = acc
        pltpu.sync_copy(accv, part_hbm.at[wid])

    return _gather_nll


def _reduce_body(p_ref, out_ref):
    out_ref[0, 0] = jnp.sum(p_ref[...]) * (1.0 / N_POS)


_reduce = pl.pallas_call(
    _reduce_body,
    out_shape=jax.ShapeDtypeStruct((1, 1), jnp.float32),
    out_specs=pl.BlockSpec(memory_space=pltpu.SMEM),
)


def kernel(emb, w_out, x):
    d, idx_m = _prep(emb, w_out, x.astype(jnp.int32))
    partials = _get_gather_nll()(d.reshape(-1), idx_m.reshape(-1))
    return _reduce(partials)[0, 0]


# speedup vs baseline: 1.2148x; 1.0111x over previous
import functools

import jax
import jax.numpy as jnp
from jax import lax
from jax.experimental import pallas as pl
from jax.experimental.pallas import tpu as pltpu
from jax.experimental.pallas import tpu_sc as plsc

VOCAB = 1000
D_MODEL = 1024
BATCH = 16
SEQ = 2048
N_POS = BATCH * (SEQ - 1)
N_PAD = BATCH * SEQ

NC, NS, L = 1, 16, 16
NW = NC * NS
PER_W = N_PAD // NW
N_GATHER = PER_W // 128


K1_GRID = 2
K1_RB = 512


def _prep_body(emb_ref, w_ref, x_ref, d_ref, im_ref):
    a = emb_ref[...].astype(jnp.bfloat16)
    b = w_ref[...].astype(jnp.bfloat16)
    m = jnp.dot(a, b, preferred_element_type=jnp.float32)
    mx = jnp.max(m, axis=1, keepdims=True)
    s = jnp.sum(jnp.exp(m - mx), axis=1, keepdims=True)
    lse = jnp.log(s) + mx
    d_ref[...] = lse - m

    @pl.when(pl.program_id(0) == 0)
    def _():
        xv = x_ref[...]
        t = jnp.concatenate([xv[:, 1:], xv[:, :1]], axis=1)
        im_ref[...] = xv * VOCAB + t


_prep = pl.pallas_call(
    _prep_body,
    grid=(K1_GRID,),
    in_specs=[
        pl.BlockSpec((K1_RB, D_MODEL), lambda i: (i, 0)),
        pl.BlockSpec((D_MODEL, VOCAB), lambda i: (0, 0)),
        pl.BlockSpec((BATCH, SEQ), lambda i: (0, 0)),
    ],
    out_specs=(
        pl.BlockSpec((K1_RB, VOCAB), lambda i: (i, 0)),
        pl.BlockSpec((BATCH, SEQ), lambda i: (0, 0)),
    ),
    out_shape=(
        jax.ShapeDtypeStruct((VOCAB, VOCAB), jnp.float32),
        jax.ShapeDtypeStruct((BATCH, SEQ), jnp.int32),
    ),
)


@functools.cache
def _get_gather_nll():
    mesh = plsc.VectorSubcoreMesh(
        core_axis_name="c", subcore_axis_name="s", num_cores=NC)

    @functools.partial(
        pl.kernel,
        mesh=mesh,
        out_type=jax.ShapeDtypeStruct((NW, L), jnp.float32),
        scratch_types=[
            pltpu.VMEM((PER_W,), jnp.int32),
            pltpu.VMEM((PER_W,), jnp.float32),
            pltpu.VMEM((L,), jnp.float32),
            pltpu.SemaphoreType.DMA,
        ],
    )
    def _gather_nll(d_hbm, im_hbm, part_hbm, idx_m, val_m, accv, sem):
        wid = lax.axis_index("s") * NC + lax.axis_index("c")
        base = wid * PER_W
        pltpu.sync_copy(im_hbm.at[pl.ds(base, PER_W)], idx_m)

        copies = []
        for j in range(N_GATHER):
            sl = pl.ds(j * 128, 128)
            copies.append(
                pltpu.async_copy(d_hbm.at[idx_m.at[sl]], val_m.at[sl], sem))
        for cp in copies:
            cp.wait()

        def accum(c, acc):
            o = c * L
            gid = base + o + lax.iota(jnp.int32, L)
            d = val_m[pl.ds(o, L)]
            return acc + jnp.where((gid & (SEQ - 1)) != SEQ - 1, d, 0.0)

        acc = lax.fori_loop(0, PER_W // L, accum, jnp.zeros((L,), jnp.float32))
        accv[...] = acc
        pltpu.sync_copy(accv, part_hbm.at[wid])

    return _gather_nll


def _reduce_body(p_ref, out_ref):
    out_ref[0, 0] = jnp.sum(p_ref[...]) * (1.0 / N_POS)


_reduce = pl.pallas_call(
    _reduce_body,
    out_shape=jax.ShapeDtypeStruct((1, 1), jnp.float32),
    out_specs=pl.BlockSpec(memory_space=pltpu.SMEM),
)


def kernel(emb, w_out, x):
    d, idx_m = _prep(emb, w_out, x.astype(jnp.int32))
    partials = _get_gather_nll()(d.reshape(-1), idx_m.reshape(-1))
    return _reduce(partials)[0, 0]
